# baseline (speedup 1.0000x reference)
import jax
import jax.numpy as jnp
from jax import lax
from jax.experimental import pallas as pl
from jax.experimental.pallas import tpu as pltpu
from jax.experimental.pallas import tpu_sc as plsc

B = 16384
DM = 60
DC = 20
NC = 2
NS = 16
NW = NC * NS
NCHUNK = 2
BC = B // NCHUNK
BPW = BC // NW
KG = 8
NG = BPW // KG


def _gather_body(m_idx, c_idx, emb_m, ecT, out_m, outcT,
                 mi_v, ci_v, mbuf, cbufT, slots, semm, semc0, semc1):
    wid = lax.axis_index("s") * NC + lax.axis_index("c")
    base = wid * BPW
    pltpu.sync_copy(m_idx.at[pl.ds(base, BPW)], mi_v.at[pl.ds(0, BPW)])
    pltpu.sync_copy(c_idx.at[pl.ds(base, BPW)], ci_v.at[pl.ds(0, BPW)])

    lanes = lax.iota(jnp.int32, 16)
    hi_mask = lanes < (DC - 16)
    semcs = (semc0, semc1)

    def fire(sset, g):
        g0 = g * KG
        vm = mi_v[pl.ds(g0, 16)]
        vc = ci_v[pl.ds(g0, 16)]
        semc = semcs[sset]
        for j in range(KG):
            pltpu.async_copy(emb_m.at[vm[j]], mbuf.at[g0 + j], semm)
            tile = pl.multiple_of(
                (lax.shift_right_logical(vc[j], 7)) * 128, 128)
            pltpu.async_copy(ecT.at[:, pl.ds(tile, 128)],
                             slots.at[sset, j], semc)

    def drain(sset):
        semc = semcs[sset]
        for j in range(KG):
            pltpu.make_async_copy(emb_m.at[0], mbuf.at[j], semm).wait()
            pltpu.make_async_copy(ecT.at[:, pl.ds(0, 128)],
                                  slots.at[sset, j], semc).wait()

    def extract(sset, g):
        g0 = g * KG
        vc = ci_v[pl.ds(g0, 16)]
        for j in range(KG):
            col = jnp.broadcast_to(lax.bitwise_and(vc[j], 127), (16,))
            bcol = jnp.broadcast_to(g0 + j, (16,))
            lo = plsc.load_gather(slots.at[sset, j], [lanes, col])
            hi = plsc.load_gather(slots.at[sset, j],
                                  [jnp.minimum(lanes + 16, DC - 1), col])
            plsc.store_scatter(cbufT, [lanes, bcol], lo)
            plsc.store_scatter(cbufT, [jnp.minimum(lanes + 16, DC - 1), bcol],
                               hi, mask=hi_mask)

    fire(0, 0)

    def pair_body(p, _):
        g = 2 * p
        fire(1, g + 1)
        drain(0)
        extract(0, g)
        fire(0, g + 2)
        drain(1)
        extract(1, g + 1)
        return 0

    lax.fori_loop(0, NG // 2 - 1, pair_body, 0)
    fire(1, NG - 1)
    drain(0)
    extract(0, NG - 2)
    drain(1)
    extract(1, NG - 1)
    pltpu.sync_copy(mbuf, out_m.at[pl.ds(base, BPW)])
    pltpu.sync_copy(cbufT, outcT.at[:, pl.ds(pl.multiple_of(base, 128), BPW)])


_gather = pl.kernel(
    _gather_body,
    out_type=(jax.ShapeDtypeStruct((BC, DM), jnp.float32),
              jax.ShapeDtypeStruct((DC, BC), jnp.float32)),
    mesh=plsc.VectorSubcoreMesh(core_axis_name="c", subcore_axis_name="s",
                                num_cores=NC, num_subcores=NS),
    scratch_types=[
        pltpu.VMEM((BPW + 16,), jnp.int32),
        pltpu.VMEM((BPW + 16,), jnp.int32),
        pltpu.VMEM((BPW, DM), jnp.float32),
        pltpu.VMEM((DC, BPW), jnp.float32),
        pltpu.VMEM((2, KG, DC, 128), jnp.float32),
        pltpu.SemaphoreType.DMA,
        pltpu.SemaphoreType.DMA,
        pltpu.SemaphoreType.DMA,
    ],
    compiler_params=pltpu.CompilerParams(needs_layout_passes=False),
)


def _sigmoid(x):
    return 1.0 / (1.0 + jnp.exp(-x))


def _bq(x):
    return x.astype(jnp.bfloat16).astype(jnp.float32)


def _mlp_body(xm, xcT, w1m, w1c, b1, w2, b2, w3, b3, w4, b4, out):
    hp = lax.Precision.HIGHEST
    h = jnp.dot(_bq(xm[...]), _bq(w1m[...]), preferred_element_type=jnp.float32,
                precision=hp)
    h += lax.dot_general(_bq(xcT[...]), _bq(w1c[...]), (((0,), (0,)), ((), ())),
                         preferred_element_type=jnp.float32, precision=hp)
    h = _bq(_sigmoid(h + b1[...]))
    h = _bq(_sigmoid(jnp.dot(h, _bq(w2[...]), preferred_element_type=jnp.float32,
                             precision=hp) + b2[...]))
    h = _bq(_sigmoid(jnp.dot(h, _bq(w3[...]), preferred_element_type=jnp.float32,
                             precision=hp) + b3[...]))
    out[...] = jnp.dot(h, _bq(w4[...]), preferred_element_type=jnp.float32,
                       precision=hp) + b4[...]


BB = 4096


def _mlp(xm, xc, w1m, w1c, b1, w2, b2, w3, b3, w4, b4):
    fixed = lambda i: (0, 0)
    return pl.pallas_call(
        _mlp_body,
        grid=(BC // BB,),
        in_specs=[
            pl.BlockSpec((BB, DM), lambda i: (i, 0)),
            pl.BlockSpec((DC, BB), lambda i: (0, i)),
            pl.BlockSpec((DM, 64), fixed),
            pl.BlockSpec((DC, 64), fixed),
            pl.BlockSpec((1, 64), fixed),
            pl.BlockSpec((64, 64), fixed),
            pl.BlockSpec((1, 64), fixed),
            pl.BlockSpec((64, 64), fixed),
            pl.BlockSpec((1, 64), fixed),
            pl.BlockSpec((64, 1), fixed),
            pl.BlockSpec((1, 1), fixed),
        ],
        out_specs=pl.BlockSpec((BB, 1), lambda i: (i, 0)),
        out_shape=jax.ShapeDtypeStruct((BC, 1), jnp.float32),
    )(xm, xc, w1m, w1c, b1, w2, b2, w3, b3, w4, b4)


def kernel(movie, consumer, emb_movie, emb_consumer,
           W1, b1, W2, b2, W3, b3, W4, b4):
    m_idx = movie.reshape(-1)
    c_idx = consumer.reshape(-1)
    ecT = emb_consumer.T
    gathered = [
        _gather(m_idx[k * BC:(k + 1) * BC], c_idx[k * BC:(k + 1) * BC],
                emb_movie, ecT)
        for k in range(NCHUNK)
    ]
    outs = [
        _mlp(xm, ocT, W1[:DM], W1[DM:], b1.reshape(1, 64),
             W2, b2.reshape(1, 64), W3, b3.reshape(1, 64),
             W4, b4.reshape(1, 1))
        for xm, ocT in gathered
    ]
    return jnp.concatenate(outs, axis=0)


# speedup vs baseline: 1.1826x; 1.1826x over previous
import jax
import jax.numpy as jnp
from jax import lax
from jax.experimental import pallas as pl
from jax.experimental.pallas import tpu as pltpu
from jax.experimental.pallas import tpu_sc as plsc

B = 16384
DM = 60
DC = 20
NC = 2
NS = 16
NW = NC * NS
NCHUNK = 2
BC = B // NCHUNK
BPW = BC // NW
KG = 8
NG = BPW // KG


def _gather_body(m_idx, c_idx, emb_m, ecT, out_m, outcT,
                 mi_v, ci_v, mbuf, cbufT, slots, semm, semc0, semc1):
    wid = lax.axis_index("s") * NC + lax.axis_index("c")
    base = wid * BPW
    pltpu.sync_copy(m_idx.at[pl.ds(base, BPW)], mi_v.at[pl.ds(0, BPW)])
    pltpu.sync_copy(c_idx.at[pl.ds(base, BPW)], ci_v.at[pl.ds(0, BPW)])

    lanes = lax.iota(jnp.int32, 16)
    hi_mask = lanes < (DC - 16)
    semcs = (semc0, semc1)

    def fire(sset, g):
        g0 = g * KG
        vm = mi_v[pl.ds(g0, 16)]
        vc = ci_v[pl.ds(g0, 16)]
        semc = semcs[sset]
        for j in range(KG):
            pltpu.async_copy(emb_m.at[vm[j]], mbuf.at[g0 + j], semm)
            tile = pl.multiple_of(
                (lax.shift_right_logical(vc[j], 7)) * 128, 128)
            pltpu.async_copy(ecT.at[:, pl.ds(tile, 128)],
                             slots.at[sset, j], semc)

    def drain(sset):
        semc = semcs[sset]
        for j in range(KG):
            pltpu.make_async_copy(emb_m.at[0], mbuf.at[j], semm).wait()
            pltpu.make_async_copy(ecT.at[:, pl.ds(0, 128)],
                                  slots.at[sset, j], semc).wait()

    def extract(sset, g):
        g0 = g * KG
        vc = ci_v[pl.ds(g0, 16)]
        for j in range(KG):
            col = jnp.broadcast_to(lax.bitwise_and(vc[j], 127), (16,))
            bcol = jnp.broadcast_to(g0 + j, (16,))
            lo = plsc.load_gather(slots.at[sset, j], [lanes, col])
            hi = plsc.load_gather(slots.at[sset, j],
                                  [jnp.minimum(lanes + 16, DC - 1), col])
            plsc.store_scatter(cbufT, [lanes, bcol], lo)
            plsc.store_scatter(cbufT, [jnp.minimum(lanes + 16, DC - 1), bcol],
                               hi, mask=hi_mask)

    fire(0, 0)

    def pair_body(p, _):
        g = 2 * p
        fire(1, g + 1)
        drain(0)
        extract(0, g)
        fire(0, g + 2)
        drain(1)
        extract(1, g + 1)
        return 0

    lax.fori_loop(0, NG // 2 - 1, pair_body, 0)
    fire(1, NG - 1)
    drain(0)
    extract(0, NG - 2)
    drain(1)
    extract(1, NG - 1)
    pltpu.sync_copy(mbuf, out_m.at[pl.ds(base, BPW)])
    pltpu.sync_copy(cbufT, outcT.at[:, pl.ds(pl.multiple_of(base, 128), BPW)])


_gather = pl.kernel(
    _gather_body,
    out_type=(jax.ShapeDtypeStruct((BC, DM), jnp.float32),
              jax.ShapeDtypeStruct((DC, BC), jnp.float32)),
    mesh=plsc.VectorSubcoreMesh(core_axis_name="c", subcore_axis_name="s",
                                num_cores=NC, num_subcores=NS),
    scratch_types=[
        pltpu.VMEM((BPW + 16,), jnp.int32),
        pltpu.VMEM((BPW + 16,), jnp.int32),
        pltpu.VMEM((BPW, DM), jnp.float32),
        pltpu.VMEM((DC, BPW), jnp.float32),
        pltpu.VMEM((2, KG, DC, 128), jnp.float32),
        pltpu.SemaphoreType.DMA,
        pltpu.SemaphoreType.DMA,
        pltpu.SemaphoreType.DMA,
    ],
    compiler_params=pltpu.CompilerParams(needs_layout_passes=False),
)


def _sigmoid(x):
    return 1.0 / (1.0 + jnp.exp(-x))


def _bq(x):
    return x.astype(jnp.bfloat16)


def _mlp_body(xm, xcT, w1m, w1c, b1, w2, b2, w3, b3, w4, b4, out):
    hp = lax.Precision.DEFAULT
    h = jnp.dot(_bq(xm[...]), _bq(w1m[...]), preferred_element_type=jnp.float32,
                precision=hp)
    h += lax.dot_general(_bq(xcT[...]), _bq(w1c[...]), (((0,), (0,)), ((), ())),
                         preferred_element_type=jnp.float32, precision=hp)
    h = _bq(_sigmoid(h + b1[...]))
    h = _bq(_sigmoid(jnp.dot(h, _bq(w2[...]), preferred_element_type=jnp.float32,
                             precision=hp) + b2[...]))
    h = _bq(_sigmoid(jnp.dot(h, _bq(w3[...]), preferred_element_type=jnp.float32,
                             precision=hp) + b3[...]))
    out[...] = jnp.dot(h, _bq(w4[...]), preferred_element_type=jnp.float32,
                       precision=hp) + b4[...]


BB = 4096


def _mlp(xm, xc, w1m, w1c, b1, w2, b2, w3, b3, w4, b4):
    fixed = lambda i: (0, 0)
    return pl.pallas_call(
        _mlp_body,
        grid=(BC // BB,),
        in_specs=[
            pl.BlockSpec((BB, DM), lambda i: (i, 0)),
            pl.BlockSpec((DC, BB), lambda i: (0, i)),
            pl.BlockSpec((DM, 64), fixed),
            pl.BlockSpec((DC, 64), fixed),
            pl.BlockSpec((1, 64), fixed),
            pl.BlockSpec((64, 64), fixed),
            pl.BlockSpec((1, 64), fixed),
            pl.BlockSpec((64, 64), fixed),
            pl.BlockSpec((1, 64), fixed),
            pl.BlockSpec((64, 1), fixed),
            pl.BlockSpec((1, 1), fixed),
        ],
        out_specs=pl.BlockSpec((BB, 1), lambda i: (i, 0)),
        out_shape=jax.ShapeDtypeStruct((BC, 1), jnp.float32),
    )(xm, xc, w1m, w1c, b1, w2, b2, w3, b3, w4, b4)


def kernel(movie, consumer, emb_movie, emb_consumer,
           W1, b1, W2, b2, W3, b3, W4, b4):
    m_idx = movie.reshape(-1)
    c_idx = consumer.reshape(-1)
    ecT = emb_consumer.T
    gathered = [
        _gather(m_idx[k * BC:(k + 1) * BC], c_idx[k * BC:(k + 1) * BC],
                emb_movie, ecT)
        for k in range(NCHUNK)
    ]
    outs = [
        _mlp(xm, ocT, W1[:DM], W1[DM:], b1.reshape(1, 64),
             W2, b2.reshape(1, 64), W3, b3.reshape(1, 64),
             W4, b4.reshape(1, 1))
        for xm, ocT in gathered
    ]
    return jnp.concatenate(outs, axis=0)
